# baseline (speedup 1.0000x reference)
import functools

import jax
import jax.numpy as jnp
from jax import lax
from jax.experimental import pallas as pl
from jax.experimental.pallas import tpu as pltpu
from jax.experimental.pallas import tpu_sc as plsc

_BATCH = 4096
_VOCAB = 100000
_DIM = 64
_NC = 2
_NS = 16
_NW = _NC * _NS
_RPW = _DIM // _NW
_LANES = 16
_GROUPS = _BATCH // _LANES
_SIZES = (56320, 28160, 14080, 1440)
_OFFS = (0, 56320, 84480, 98560)
_LCAP = _BATCH + _LANES
_POSBITS = 13
_POSMASK = (1 << _POSBITS) - 1


@functools.partial(
    pl.kernel,
    mesh=plsc.VectorSubcoreMesh(core_axis_name="c", subcore_axis_name="s"),
    out_type=jax.ShapeDtypeStruct((_DIM, _BATCH), jnp.float32),
    scratch_types=[
        pltpu.VMEM((_BATCH,), jnp.int32),
        pltpu.VMEM((_SIZES[0],), jnp.float32),
        pltpu.VMEM((_SIZES[1],), jnp.float32),
        pltpu.VMEM((_SIZES[2],), jnp.float32),
        pltpu.VMEM((_SIZES[3],), jnp.float32),
        pltpu.VMEM((_LCAP,), jnp.int32),
        pltpu.VMEM((_LCAP,), jnp.int32),
        pltpu.VMEM((_LCAP,), jnp.int32),
        pltpu.VMEM((_LCAP,), jnp.float32),
        pltpu.VMEM((_LCAP,), jnp.float32),
        pltpu.SemaphoreType.DMA,
        pltpu.SemaphoreType.DMA,
        pltpu.SemaphoreType.DMA,
        pltpu.SemaphoreType.DMA,
    ],
    compiler_params=pltpu.CompilerParams(needs_layout_passes=False),
)
def _gather_add(xt_hbm, tablet_hbm, idx_hbm, outt_hbm,
                idx_v, buf0, buf1, buf2, buf3, list1, list2, list3,
                x_v, o_v, sem0, sem1, sem2, sem3):
    wid = lax.axis_index("s") * _NC + lax.axis_index("c")
    c0row = wid * _RPW
    bufs = (buf0, buf1, buf2, buf3)
    lists = (None, list1, list2, list3)
    sems = (sem0, sem1, sem2, sem3)

    def issue(row, j):
        return pltpu.async_copy(
            tablet_hbm.at[row, pl.ds(_OFFS[j], _SIZES[j])], bufs[j], sems[j])

    cps = [issue(c0row, j) for j in range(4)]
    pltpu.sync_copy(idx_hbm, idx_v)

    lane = lax.iota(jnp.int32, _LANES)

    def count(mask):
        pc = plsc.all_reduce_population_count(mask)
        return jnp.squeeze(lax.slice(pc, (0,), (1,)))

    def part_body(g, ns):
        n1, n2, n3 = ns
        sl = pl.ds(g * _LANES, _LANES)
        iv = idx_v[sl]
        key = (iv << _POSBITS) | (g * _LANES + lane)
        ge1 = iv >= _OFFS[1]
        ge2 = iv >= _OFFS[2]
        ge3 = iv >= _OFFS[3]
        m1 = jnp.logical_and(ge1, jnp.logical_not(ge2))
        m2 = jnp.logical_and(ge2, jnp.logical_not(ge3))
        plsc.store_compressed(list1.at[pl.ds(n1, _LANES)], key, mask=m1)
        plsc.store_compressed(list2.at[pl.ds(n2, _LANES)], key, mask=m2)
        plsc.store_compressed(list3.at[pl.ds(n3, _LANES)], key, mask=ge3)
        return (n1 + count(m1), n2 + count(m2), n3 + count(ge3))

    zero = jnp.int32(0)
    ns = lax.fori_loop(0, _GROUPS, part_body, (zero, zero, zero))
    for j in (1, 2, 3):
        pad = jnp.full((_LANES,), (_OFFS[j] << _POSBITS) | _BATCH, jnp.int32)
        lists[j][pl.ds(ns[j - 1], _LANES)] = pad

    def pass0(g, carry):
        sl = pl.ds(g * _LANES, _LANES)
        i0 = jnp.minimum(idx_v[sl], _SIZES[0] - 1)
        o_v[sl] = x_v[sl] + plsc.load_gather(buf0, [i0])
        return carry

    def make_pass(j):
        lst = lists[j]

        def pass_body(g, carry):
            sl = pl.ds(g * _LANES, _LANES)
            k = lst[sl]
            p = k & _POSMASK
            i = lax.shift_right_logical(k, _POSBITS) - _OFFS[j]
            v = plsc.load_gather(bufs[j], [i])
            xv = plsc.load_gather(x_v, [p])
            plsc.store_scatter(o_v, [p], xv + v)
            return carry
        return pass_body

    passes = (pass0, make_pass(1), make_pass(2), make_pass(3))

    for t in range(_RPW):
        row = c0row + t
        pltpu.sync_copy(xt_hbm.at[row], x_v.at[pl.ds(0, _BATCH)])
        for j in range(4):
            cps[j].wait()
            if j == 0:
                lax.fori_loop(0, _GROUPS, pass0, 0)
            else:
                gmax = (ns[j - 1] + _LANES - 1) // _LANES
                lax.fori_loop(0, gmax, passes[j], 0)
            if t + 1 < _RPW:
                cps[j] = issue(row + 1, j)
        pltpu.sync_copy(o_v.at[pl.ds(0, _BATCH)], outt_hbm.at[row])


def kernel(x, const, indices):
    out_t = _gather_add(x.T, const.T, indices.astype(jnp.int32))
    return out_t.T


# speedup vs baseline: 1.1555x; 1.1555x over previous
import functools

import jax
import jax.numpy as jnp
from jax import lax
from jax.experimental import pallas as pl
from jax.experimental.pallas import tpu as pltpu
from jax.experimental.pallas import tpu_sc as plsc

_BATCH = 4096
_VOCAB = 100000
_DIM = 64
_NC = 2
_NS = 16
_NW = _NC * _NS
_RPW = _DIM // _NW
_LANES = 16
_H0 = 50048
_H1 = _VOCAB - _H0
_GROUPS = _BATCH // _LANES


@functools.partial(
    pl.kernel,
    mesh=plsc.VectorSubcoreMesh(core_axis_name="c", subcore_axis_name="s"),
    out_type=jax.ShapeDtypeStruct((_DIM, _BATCH), jnp.float32),
    scratch_types=[
        pltpu.VMEM((_BATCH,), jnp.int32),
        pltpu.VMEM((_H0,), jnp.float32),
        pltpu.VMEM((_H1,), jnp.float32),
        pltpu.VMEM((_BATCH,), jnp.float32),
        pltpu.VMEM((_BATCH,), jnp.float32),
        pltpu.VMEM((_BATCH,), jnp.float32),
        pltpu.VMEM((_BATCH,), jnp.float32),
        pltpu.VMEM((_BATCH,), jnp.float32),
        pltpu.SemaphoreType.DMA,
        pltpu.SemaphoreType.DMA,
        pltpu.SemaphoreType.DMA,
        pltpu.SemaphoreType.DMA,
    ],
    compiler_params=pltpu.CompilerParams(needs_layout_passes=False),
)
def _gather_add(xt_hbm, tablet_hbm, idx_hbm, outt_hbm,
                idx_v, buf0, buf1, tmp_v, x_v0, x_v1, o_v0, o_v1,
                semA, semB, semX, semO):
    wid = lax.axis_index("s") * _NC + lax.axis_index("c")
    c0 = wid * _RPW
    x_vs = (x_v0, x_v1)
    o_vs = (o_v0, o_v1)

    cpA = pltpu.async_copy(tablet_hbm.at[c0, pl.ds(0, _H0)], buf0, semA)
    cpB = pltpu.async_copy(tablet_hbm.at[c0, pl.ds(_H0, _H1)], buf1, semB)
    cpXs = [pltpu.async_copy(xt_hbm.at[c0 + t], x_vs[t], semX)
            for t in range(_RPW)]
    pltpu.sync_copy(idx_hbm, idx_v)

    def pass_low(g, carry):
        sl = pl.ds(g * _LANES, _LANES)
        i0 = jnp.minimum(idx_v[sl], _H0 - 1)
        tmp_v[sl] = plsc.load_gather(buf0, [i0])
        return carry

    def make_pass_high(x_v, o_v):
        def pass_high(g, carry):
            sl = pl.ds(g * _LANES, _LANES)
            iv = idx_v[sl]
            i1 = jnp.minimum(jnp.maximum(iv, _H0) - _H0, _H1 - 1)
            v1 = plsc.load_gather(buf1, [i1])
            o_v[sl] = x_v[sl] + jnp.where(iv < _H0, tmp_v[sl], v1)
            return carry
        return pass_high

    cpOs = []
    for t in range(_RPW):
        c = c0 + t
        cpA.wait()
        lax.fori_loop(0, _GROUPS, pass_low, 0)
        if t + 1 < _RPW:
            cpA = pltpu.async_copy(
                tablet_hbm.at[c + 1, pl.ds(0, _H0)], buf0, semA)
        cpXs[t].wait()
        cpB.wait()
        lax.fori_loop(0, _GROUPS, make_pass_high(x_vs[t], o_vs[t]), 0)
        if t + 1 < _RPW:
            cpB = pltpu.async_copy(
                tablet_hbm.at[c + 1, pl.ds(_H0, _H1)], buf1, semB)
        cpOs.append(pltpu.async_copy(o_vs[t], outt_hbm.at[c], semO))
    for cp in cpOs:
        cp.wait()


def kernel(x, const, indices):
    out_t = _gather_add(x.T, const.T, indices.astype(jnp.int32))
    return out_t.T
